# baseline (speedup 1.0000x reference)
import functools
import jax
import jax.numpy as jnp
from jax import lax
from jax.experimental import pallas as pl
from jax.experimental.pallas import tpu as pltpu
from jax.experimental.pallas import tpu_sc as plsc

B = 4096
NB = 200
D = 100
L = 16
NBP = 208
DP = 112
TW = 128
NW = 32
RPW = B // NW
GRP = 4
NGRP = RPW // GRP

_NEG = -1e9

_GDN = lax.GatherDimensionNumbers(offset_dims=(), collapsed_slice_dims=(0,),
                                  start_index_map=(0,))


def _take16(vec, j):
    idx = jnp.full((L, 1), j, jnp.int32)
    return lax.gather(vec, idx, _GDN, (1,),
                      mode=lax.GatherScatterMode.PROMISE_IN_BOUNDS)


def _sc_body(sym, relidx, entidx, selfidx_h, nnb_h, u1_h, u2_h,
             r_out, e_out, self_out,
             relidx_v, entidx_v, nb_v, selfidx_v, self_rows,
             rel_rows, ent_rows, r_out_v, e_out_v, u1_v, u2_v,
             logits_v, w_v,
             sem_r0, sem_r1, sem_e0, sem_e1, sem_self):
    wid = lax.axis_index("s") * 2 + lax.axis_index("c")
    base = wid * RPW
    pltpu.sync_copy(u1_h, u1_v)
    pltpu.sync_copy(u2_h, u2_v)
    pltpu.sync_copy(nnb_h.at[pl.ds(base, RPW)], nb_v)
    pltpu.sync_copy(selfidx_h.at[pl.ds(base, RPW)], selfidx_v)

    zero16 = jnp.zeros((L,), jnp.float32)
    for rows in (rel_rows, ent_rows):
        for bb in range(2):
            for rr in range(NBP - NB):
                for k in range(DP // L):
                    rows[bb, NB + rr, pl.ds(k * L, L)] = zero16

    sem_r = (sem_r0, sem_r1)
    sem_e = (sem_e0, sem_e1)

    def gather_cp(r, buf):
        cs = []
        for h in range(2):
            cs.append(pltpu.make_async_copy(
                sym.at[relidx_v.at[r, h]],
                rel_rows.at[buf, pl.ds(h * 100, 100)], sem_r[buf]))
            cs.append(pltpu.make_async_copy(
                sym.at[entidx_v.at[r, h]],
                ent_rows.at[buf, pl.ds(h * 100, 100)], sem_e[buf]))
        return cs

    def issue(r, buf):
        for cp in gather_cp(r, buf):
            cp.start()

    def wait(r, buf):
        for cp in gather_cp(r, buf):
            cp.wait()

    iota16 = lax.iota(jnp.int32, L)
    cvec = u1_v[pl.ds(DP, L)]

    def group_body(g, carry):
        gbase = base + g * GRP
        pltpu.sync_copy(relidx.at[pl.ds(gbase, GRP)], relidx_v)
        pltpu.sync_copy(entidx.at[pl.ds(gbase, GRP)], entidx_v)
        issue(0, 0)

        def compute_row(r, buf):
            row_local = g * GRP + r
            nbch = nb_v[pl.ds((row_local // L) * L, L)]
            nbvec = _take16(nbch, row_local % L)
            nb = nbvec[0]
            nch = (nb + (L - 1)) >> 4
            bufsplat = jnp.full((L,), buf, jnp.int32)

            def p1_chunk(c, mmax):
                rowidx = iota16 + c * L
                acc = [zero16] * 4
                for k in range(7):
                    u1k = u1_v[pl.ds(k * L, L)]
                    u2k = u2_v[pl.ds(k * L, L)]
                    for j in range(L):
                        d = k * L + j
                        if d >= D:
                            break
                        dsplat = jnp.full((L,), d, jnp.int32)
                        colr = plsc.load_gather(rel_rows, [bufsplat, rowidx, dsplat])
                        cole = plsc.load_gather(ent_rows, [bufsplat, rowidx, dsplat])
                        s = d & 1
                        acc[s] = acc[s] + colr * _take16(u1k, j)
                        acc[2 + s] = acc[2 + s] + cole * _take16(u2k, j)
                lvec = ((acc[0] + acc[1]) + (acc[2] + acc[3])) + cvec
                lvec = jnp.where(lvec > 0, lvec, 0.2 * lvec)
                lvec = jnp.where(rowidx < nb, lvec, _NEG)
                logits_v[pl.ds(c * L, L)] = lvec
                return jnp.maximum(mmax, lvec)

            mmax = lax.fori_loop(0, nch, p1_chunk, jnp.full((L,), _NEG, jnp.float32))
            mvec = _take16(plsc.cummax(mmax), L - 1)

            def pexp_chunk(c, ssum):
                lv = logits_v[pl.ds(c * L, L)]
                wv = jnp.exp(lv - mvec)
                w_v[pl.ds(c * L, L)] = wv
                return ssum + wv

            ssum = lax.fori_loop(0, nch, pexp_chunk, zero16)
            svec = _take16(plsc.cumsum(ssum), L - 1)

            def p2_chunk(c, accs):
                raccs, eaccs = accs
                wv = w_v[pl.ds(c * L, L)]
                nr = list(raccs)
                ne = list(eaccs)
                for j in range(L):
                    wj = _take16(wv, j)
                    rsplat = jnp.full((L,), c * L + j, jnp.int32)
                    for k in range(7):
                        colv = iota16 + k * L
                        rch = plsc.load_gather(rel_rows, [bufsplat, rsplat, colv])
                        ech = plsc.load_gather(ent_rows, [bufsplat, rsplat, colv])
                        nr[k] = nr[k] + wj * rch
                        ne[k] = ne[k] + wj * ech
                return tuple(nr), tuple(ne)

            zaccs = tuple(zero16 for _ in range(7))
            raccs, eaccs = lax.fori_loop(0, nch, p2_chunk, (zaccs, zaccs))
            for k in range(7):
                r_out_v[r, pl.ds(k * L, L)] = raccs[k] / svec
                e_out_v[r, pl.ds(k * L, L)] = eaccs[k] / svec

        def pair_body(p, c2):
            for parity in range(2):
                r = p * 2 + parity
                if parity == 0:
                    issue(r + 1, 1)
                else:
                    @pl.when(p < GRP // 2 - 1)
                    def _():
                        issue(r + 1, 0)
                wait(r, parity)
                compute_row(r, parity)
            return c2

        lax.fori_loop(0, GRP // 2, pair_body, 0)
        pltpu.sync_copy(r_out_v, r_out.at[pl.ds(gbase, GRP)])
        pltpu.sync_copy(e_out_v, e_out.at[pl.ds(gbase, GRP)])
        return carry

    lax.fori_loop(0, NGRP, group_body, 0)
    for h in range(16):
        cp = pltpu.make_async_copy(
            sym.at[selfidx_v.at[pl.ds(h * 8, 8)]], self_rows, sem_self)
        cp.start()
        cp.wait()
        pltpu.sync_copy(self_rows, self_out.at[pl.ds(base + h * 8, 8)])


_sc_mesh = plsc.VectorSubcoreMesh(core_axis_name="c", subcore_axis_name="s")

_sc_call = pl.kernel(
    _sc_body, mesh=_sc_mesh,
    compiler_params=pltpu.CompilerParams(needs_layout_passes=False,
                                         use_tc_tiling_on_sc=False),
    out_type=[
        jax.ShapeDtypeStruct((B, DP), jnp.float32),
        jax.ShapeDtypeStruct((B, DP), jnp.float32),
        jax.ShapeDtypeStruct((B, TW), jnp.float32),
    ],
    scratch_types=[
        pltpu.VMEM((GRP, 2, 100), jnp.int32),
        pltpu.VMEM((GRP, 2, 100), jnp.int32),
        pltpu.VMEM((RPW,), jnp.int32),
        pltpu.VMEM((RPW,), jnp.int32),
        pltpu.VMEM((8, TW), jnp.float32),
        pltpu.VMEM((2, NBP, TW), jnp.float32),
        pltpu.VMEM((2, NBP, TW), jnp.float32),
        pltpu.VMEM((GRP, DP), jnp.float32),
        pltpu.VMEM((GRP, DP), jnp.float32),
        pltpu.VMEM((8 * L,), jnp.float32),
        pltpu.VMEM((8 * L,), jnp.float32),
        pltpu.VMEM((NBP,), jnp.float32),
        pltpu.VMEM((NBP,), jnp.float32),
        pltpu.SemaphoreType.DMA,
        pltpu.SemaphoreType.DMA,
        pltpu.SemaphoreType.DMA,
        pltpu.SemaphoreType.DMA,
        pltpu.SemaphoreType.DMA,
    ],
)


PBLK = 8192
VP1 = 500001
PGRID = (VP1 + PBLK - 1) // PBLK


def _pad_body(x, o):
    o[...] = jnp.concatenate(
        [x[...], jnp.zeros((PBLK, TW - D), jnp.float32)], axis=1)


def _pad_table(symbol_emb):
    return pl.pallas_call(
        _pad_body,
        grid=(PGRID,),
        in_specs=[pl.BlockSpec((PBLK, D), lambda i: (i, 0))],
        out_specs=pl.BlockSpec((PBLK, TW), lambda i: (i, 0)),
        out_shape=jax.ShapeDtypeStruct((VP1, TW), jnp.float32),
    )(symbol_emb)


BLK = 512


def _tc_body(r112, e112, selfe, w1p, w2p, bias, wg, cvec, out):
    bg = cvec[0, 0]
    agg = (jnp.dot(r112[...], w1p[...], preferred_element_type=jnp.float32)
           + jnp.dot(e112[...], w2p[...], preferred_element_type=jnp.float32) + bias[0])
    glog = jnp.sum(agg * wg[...], axis=1, keepdims=True)
    g = jax.nn.sigmoid(glog + bg)
    out[...] = jnp.tanh(g * agg + (1.0 - g) * selfe[..., :D])


def kernel(connections, num_neighbors, symbol_emb, W_gcn, b_gcn, gcn_b, W_attn, b_attn, W_gate, b_gate, gate_b):
    relidx = connections[:, :, 1].astype(jnp.int32).reshape(B, 2, 100)
    entidx = connections[:, :, 2].astype(jnp.int32).reshape(B, 2, 100)
    selfidx = connections[:, 0, 0].astype(jnp.int32)
    nnb = jnp.maximum(num_neighbors, 1).astype(jnp.int32)
    symp = _pad_table(symbol_emb)

    u = (W_attn @ W_gcn)[0]
    c = W_attn[0] @ (b_gcn + gcn_b) + b_attn[0]
    u1p = jnp.concatenate([u[:D], jnp.zeros((DP - D,), jnp.float32),
                           jnp.full((L,), c, jnp.float32)])
    u2p = jnp.concatenate([u[D:], jnp.zeros((28,), jnp.float32)])

    r112, e112, self128 = _sc_call(symp, relidx, entidx, selfidx, nnb, u1p, u2p)

    w1p = jnp.concatenate([W_gcn[:, :D].T, jnp.zeros((DP - D, D), jnp.float32)], axis=0)
    w2p = jnp.concatenate([W_gcn[:, D:].T, jnp.zeros((DP - D, D), jnp.float32)], axis=0)
    bias = (b_gcn + gcn_b)[None, :]
    cvec = jnp.stack([b_gate[0] + gate_b[0], 0.0])[None, :]

    grid = B // BLK
    out = pl.pallas_call(
        _tc_body,
        grid=(grid,),
        in_specs=[
            pl.BlockSpec((BLK, DP), lambda i: (i, 0)),
            pl.BlockSpec((BLK, DP), lambda i: (i, 0)),
            pl.BlockSpec((BLK, TW), lambda i: (i, 0)),
            pl.BlockSpec((DP, D), lambda i: (0, 0)),
            pl.BlockSpec((DP, D), lambda i: (0, 0)),
            pl.BlockSpec((1, D), lambda i: (0, 0)),
            pl.BlockSpec((1, D), lambda i: (0, 0)),
            pl.BlockSpec((1, 2), lambda i: (0, 0)),
        ],
        out_specs=pl.BlockSpec((BLK, D), lambda i: (i, 0)),
        out_shape=jax.ShapeDtypeStruct((B, D), jnp.float32),
    )(r112, e112, self128, w1p, w2p, bias, W_gate, cvec)
    return out


# speedup vs baseline: 2.4954x; 2.4954x over previous
import functools
import jax
import jax.numpy as jnp
from jax import lax
from jax.experimental import pallas as pl
from jax.experimental.pallas import tpu as pltpu
from jax.experimental.pallas import tpu_sc as plsc

B = 4096
NB = 200
D = 100
L = 16
NBP = 208
DP = 112
TW = 128
NW = 32
RPW = B // NW
GRP = 16
NGRP = RPW // GRP

_NEG = -1e9

_GDN = lax.GatherDimensionNumbers(offset_dims=(), collapsed_slice_dims=(0,),
                                  start_index_map=(0,))


def _take16(vec, j):
    idx = jnp.full((L, 1), j, jnp.int32)
    return lax.gather(vec, idx, _GDN, (1,),
                      mode=lax.GatherScatterMode.PROMISE_IN_BOUNDS)


def _sc_body(sym, relidx, entidx, selfidx_h, nnb_h, u1_h, u2_h,
             r_out, e_out, self_out,
             relidx_v, entidx_v, nb_v, selfidx_v, self_rows,
             rel_rows, ent_rows, r_out_v, e_out_v, u1_v, u2_v,
             logits_v, w_v,
             sem_r0, sem_r1, sem_e0, sem_e1, sem_self):
    wid = lax.axis_index("s") * 2 + lax.axis_index("c")
    base = wid * RPW
    pltpu.sync_copy(u1_h, u1_v)
    pltpu.sync_copy(u2_h, u2_v)
    pltpu.sync_copy(nnb_h.at[pl.ds(base, RPW)], nb_v)
    pltpu.sync_copy(selfidx_h.at[pl.ds(base, RPW)], selfidx_v)

    zero16 = jnp.zeros((L,), jnp.float32)
    for rows in (rel_rows, ent_rows):
        for bb in range(2):
            for rr in range(NBP - NB):
                for k in range(DP // L):
                    rows[bb, NB + rr, pl.ds(k * L, L)] = zero16

    sem_r = (sem_r0, sem_r1)
    sem_e = (sem_e0, sem_e1)

    def gather_cp(r, buf):
        cs = []
        for h in range(2):
            cs.append(pltpu.make_async_copy(
                sym.at[relidx_v.at[r, h]],
                rel_rows.at[buf, pl.ds(h * 100, 100)], sem_r[buf]))
            cs.append(pltpu.make_async_copy(
                sym.at[entidx_v.at[r, h]],
                ent_rows.at[buf, pl.ds(h * 100, 100)], sem_e[buf]))
        return cs

    def issue(r, buf):
        for cp in gather_cp(r, buf):
            cp.start()

    def wait(r, buf):
        for cp in gather_cp(r, buf):
            cp.wait()

    iota16 = lax.iota(jnp.int32, L)
    cvec = u1_v[pl.ds(DP, L)]

    def group_body(g, carry):
        gbase = base + g * GRP
        pltpu.sync_copy(relidx.at[pl.ds(gbase, GRP)], relidx_v)
        pltpu.sync_copy(entidx.at[pl.ds(gbase, GRP)], entidx_v)
        issue(0, 0)

        def compute_row(r, buf):
            row_local = g * GRP + r
            nbch = nb_v[pl.ds((row_local // L) * L, L)]
            nbvec = _take16(nbch, row_local % L)
            nb = nbvec[0]
            nch = (nb + (L - 1)) >> 4

            def p1_chunk(c, mmax):
                rowidx = iota16 + c * L
                uk1 = [u1_v[pl.ds(k * L, L)] for k in range(7)]
                uk2 = [u2_v[pl.ds(k * L, L)] for k in range(7)]
                lvec = cvec
                for j in range(L):
                    n = c * L + j
                    acc0 = zero16
                    acc1 = zero16
                    for k in range(7):
                        acc0 = acc0 + rel_rows[buf, n, pl.ds(k * L, L)] * uk1[k]
                        acc1 = acc1 + ent_rows[buf, n, pl.ds(k * L, L)] * uk2[k]
                    tot = _take16(plsc.cumsum(acc0 + acc1), L - 1)
                    lvec = jnp.where(iota16 == j, lvec + tot, lvec)
                lvec = jnp.where(lvec > 0, lvec, 0.2 * lvec)
                lvec = jnp.where(rowidx < nb, lvec, _NEG)
                logits_v[pl.ds(c * L, L)] = lvec
                return jnp.maximum(mmax, lvec)

            mmax = lax.fori_loop(0, nch, p1_chunk, jnp.full((L,), _NEG, jnp.float32))
            mvec = _take16(plsc.cummax(mmax), L - 1)

            def pexp_chunk(c, ssum):
                lv = logits_v[pl.ds(c * L, L)]
                wv = jnp.exp(lv - mvec)
                w_v[pl.ds(c * L, L)] = wv
                return ssum + wv

            ssum = lax.fori_loop(0, nch, pexp_chunk, zero16)
            svec = _take16(plsc.cumsum(ssum), L - 1)

            def p2_chunk(c, accs):
                raccs, eaccs = accs
                wv = w_v[pl.ds(c * L, L)]
                nr = list(raccs)
                ne = list(eaccs)
                for j in range(L):
                    wj = _take16(wv, j)
                    n = c * L + j
                    for k in range(7):
                        nr[k] = nr[k] + wj * rel_rows[buf, n, pl.ds(k * L, L)]
                        ne[k] = ne[k] + wj * ent_rows[buf, n, pl.ds(k * L, L)]
                return tuple(nr), tuple(ne)

            zaccs = tuple(zero16 for _ in range(7))
            raccs, eaccs = lax.fori_loop(0, nch, p2_chunk, (zaccs, zaccs))
            for k in range(7):
                r_out_v[r, pl.ds(k * L, L)] = raccs[k] / svec
                e_out_v[r, pl.ds(k * L, L)] = eaccs[k] / svec

        def pair_body(p, c2):
            for parity in range(2):
                r = p * 2 + parity
                if parity == 0:
                    issue(r + 1, 1)
                else:
                    @pl.when(p < GRP // 2 - 1)
                    def _():
                        issue(r + 1, 0)
                wait(r, parity)
                compute_row(r, parity)
            return c2

        lax.fori_loop(0, GRP // 2, pair_body, 0)
        pltpu.sync_copy(r_out_v, r_out.at[pl.ds(gbase, GRP)])
        pltpu.sync_copy(e_out_v, e_out.at[pl.ds(gbase, GRP)])
        return carry

    lax.fori_loop(0, NGRP, group_body, 0)
    for h in range(4):
        cp = pltpu.make_async_copy(
            sym.at[selfidx_v.at[pl.ds(h * 32, 32)]], self_rows, sem_self)
        cp.start()
        cp.wait()
        pltpu.sync_copy(self_rows, self_out.at[pl.ds(base + h * 32, 32)])


_sc_mesh = plsc.VectorSubcoreMesh(core_axis_name="c", subcore_axis_name="s")

_sc_call = pl.kernel(
    _sc_body, mesh=_sc_mesh,
    compiler_params=pltpu.CompilerParams(needs_layout_passes=False,
                                         use_tc_tiling_on_sc=False),
    out_type=[
        jax.ShapeDtypeStruct((B, DP), jnp.float32),
        jax.ShapeDtypeStruct((B, DP), jnp.float32),
        jax.ShapeDtypeStruct((B, TW), jnp.float32),
    ],
    scratch_types=[
        pltpu.VMEM((GRP, 2, 100), jnp.int32),
        pltpu.VMEM((GRP, 2, 100), jnp.int32),
        pltpu.VMEM((RPW,), jnp.int32),
        pltpu.VMEM((RPW,), jnp.int32),
        pltpu.VMEM((32, TW), jnp.float32),
        pltpu.VMEM((2, NBP, TW), jnp.float32),
        pltpu.VMEM((2, NBP, TW), jnp.float32),
        pltpu.VMEM((GRP, DP), jnp.float32),
        pltpu.VMEM((GRP, DP), jnp.float32),
        pltpu.VMEM((8 * L,), jnp.float32),
        pltpu.VMEM((8 * L,), jnp.float32),
        pltpu.VMEM((NBP,), jnp.float32),
        pltpu.VMEM((NBP,), jnp.float32),
        pltpu.SemaphoreType.DMA,
        pltpu.SemaphoreType.DMA,
        pltpu.SemaphoreType.DMA,
        pltpu.SemaphoreType.DMA,
        pltpu.SemaphoreType.DMA,
    ],
)


PBLK = 8192
VP1 = 500001
PGRID = (VP1 + PBLK - 1) // PBLK


def _pad_body(x, o):
    o[...] = jnp.concatenate(
        [x[...], jnp.zeros((PBLK, TW - D), jnp.float32)], axis=1)


def _pad_table(symbol_emb):
    return pl.pallas_call(
        _pad_body,
        grid=(PGRID,),
        in_specs=[pl.BlockSpec((PBLK, D), lambda i: (i, 0))],
        out_specs=pl.BlockSpec((PBLK, TW), lambda i: (i, 0)),
        out_shape=jax.ShapeDtypeStruct((VP1, TW), jnp.float32),
    )(symbol_emb)


BLK = 512


def _tc_body(r112, e112, selfe, w1p, w2p, bias, wg, cvec, out):
    bg = cvec[0, 0]
    agg = (jnp.dot(r112[...], w1p[...], preferred_element_type=jnp.float32)
           + jnp.dot(e112[...], w2p[...], preferred_element_type=jnp.float32) + bias[0])
    glog = jnp.sum(agg * wg[...], axis=1, keepdims=True)
    g = jax.nn.sigmoid(glog + bg)
    out[...] = jnp.tanh(g * agg + (1.0 - g) * selfe[..., :D])


def kernel(connections, num_neighbors, symbol_emb, W_gcn, b_gcn, gcn_b, W_attn, b_attn, W_gate, b_gate, gate_b):
    relidx = connections[:, :, 1].astype(jnp.int32).reshape(B, 2, 100)
    entidx = connections[:, :, 2].astype(jnp.int32).reshape(B, 2, 100)
    selfidx = connections[:, 0, 0].astype(jnp.int32)
    nnb = jnp.maximum(num_neighbors, 1).astype(jnp.int32)
    symp = _pad_table(symbol_emb)

    u = (W_attn @ W_gcn)[0]
    c = W_attn[0] @ (b_gcn + gcn_b) + b_attn[0]
    u1p = jnp.concatenate([u[:D], jnp.zeros((DP - D,), jnp.float32),
                           jnp.full((L,), c, jnp.float32)])
    u2p = jnp.concatenate([u[D:], jnp.zeros((28,), jnp.float32)])

    r112, e112, self128 = _sc_call(symp, relidx, entidx, selfidx, nnb, u1p, u2p)

    w1p = jnp.concatenate([W_gcn[:, :D].T, jnp.zeros((DP - D, D), jnp.float32)], axis=0)
    w2p = jnp.concatenate([W_gcn[:, D:].T, jnp.zeros((DP - D, D), jnp.float32)], axis=0)
    bias = (b_gcn + gcn_b)[None, :]
    cvec = jnp.stack([b_gate[0] + gate_b[0], 0.0])[None, :]

    grid = B // BLK
    out = pl.pallas_call(
        _tc_body,
        grid=(grid,),
        in_specs=[
            pl.BlockSpec((BLK, DP), lambda i: (i, 0)),
            pl.BlockSpec((BLK, DP), lambda i: (i, 0)),
            pl.BlockSpec((BLK, TW), lambda i: (i, 0)),
            pl.BlockSpec((DP, D), lambda i: (0, 0)),
            pl.BlockSpec((DP, D), lambda i: (0, 0)),
            pl.BlockSpec((1, D), lambda i: (0, 0)),
            pl.BlockSpec((1, D), lambda i: (0, 0)),
            pl.BlockSpec((1, 2), lambda i: (0, 0)),
        ],
        out_specs=pl.BlockSpec((BLK, D), lambda i: (i, 0)),
        out_shape=jax.ShapeDtypeStruct((B, D), jnp.float32),
    )(r112, e112, self128, w1p, w2p, bias, W_gate, cvec)
    return out


# speedup vs baseline: 2.9975x; 1.2012x over previous
import functools
import jax
import jax.numpy as jnp
from jax import lax
from jax.experimental import pallas as pl
from jax.experimental.pallas import tpu as pltpu
from jax.experimental.pallas import tpu_sc as plsc

B = 4096
NB = 200
D = 100
L = 16
NBP = 208
DP = 112
TW = 128
NW = 32
RPW = B // NW
GRP = 16
NGRP = RPW // GRP

_NEG = -1e9

_GDN = lax.GatherDimensionNumbers(offset_dims=(), collapsed_slice_dims=(0,),
                                  start_index_map=(0,))


def _take16(vec, j):
    idx = jnp.full((L, 1), j, jnp.int32)
    return lax.gather(vec, idx, _GDN, (1,),
                      mode=lax.GatherScatterMode.PROMISE_IN_BOUNDS)


def _sc_body(sym, relidx, entidx, selfidx_h, nnb_h, u1_h, u2_h,
             r_out, e_out, self_out,
             relidx_v, entidx_v, nb_v, selfidx_v, self_rows,
             rel_rows, ent_rows, r_out_v, e_out_v, u1_v, u2_v,
             logits_v, w_v,
             sem_r0, sem_r1, sem_e0, sem_e1, sem_self):
    wid = lax.axis_index("s") * 2 + lax.axis_index("c")
    base = wid * RPW
    pltpu.sync_copy(u1_h, u1_v)
    pltpu.sync_copy(u2_h, u2_v)
    pltpu.sync_copy(nnb_h.at[pl.ds(base, RPW)], nb_v)
    pltpu.sync_copy(selfidx_h.at[pl.ds(base, RPW)], selfidx_v)

    zero16 = jnp.zeros((L,), jnp.float32)
    for rows in (rel_rows, ent_rows):
        for bb in range(2):
            for rr in range(NBP - NB):
                for k in range(DP // L):
                    rows[bb, NB + rr, pl.ds(k * L, L)] = zero16

    sem_r = (sem_r0, sem_r1)
    sem_e = (sem_e0, sem_e1)

    def gather_cp(r, buf):
        cs = []
        for h in range(2):
            cs.append(pltpu.make_async_copy(
                sym.at[relidx_v.at[r, h]],
                rel_rows.at[buf, pl.ds(h * 100, 100)], sem_r[buf]))
            cs.append(pltpu.make_async_copy(
                sym.at[entidx_v.at[r, h]],
                ent_rows.at[buf, pl.ds(h * 100, 100)], sem_e[buf]))
        return cs

    def issue(r, buf):
        for cp in gather_cp(r, buf):
            cp.start()

    def wait(r, buf):
        for cp in gather_cp(r, buf):
            cp.wait()

    iota16 = lax.iota(jnp.int32, L)
    cvec = u1_v[pl.ds(DP, L)]

    def group_body(g, carry):
        gbase = base + g * GRP
        pltpu.sync_copy(relidx.at[pl.ds(gbase, GRP)], relidx_v)
        pltpu.sync_copy(entidx.at[pl.ds(gbase, GRP)], entidx_v)
        issue(0, 0)

        def compute_row(r, buf):
            row_local = g * GRP + r
            nbch = nb_v[pl.ds((row_local // L) * L, L)]
            nbvec = _take16(nbch, row_local % L)
            nb = nbvec[0]
            nch = (nb + (L - 1)) >> 4

            def p1_chunk(c, mmax):
                rowidx = iota16 + c * L
                uk1 = [u1_v[pl.ds(k * L, L)] for k in range(7)]
                uk2 = [u2_v[pl.ds(k * L, L)] for k in range(7)]
                lvec = cvec
                for j in range(L):
                    n = c * L + j
                    acc0 = zero16
                    acc1 = zero16
                    for k in range(7):
                        acc0 = acc0 + rel_rows[buf, n, pl.ds(k * L, L)] * uk1[k]
                        acc1 = acc1 + ent_rows[buf, n, pl.ds(k * L, L)] * uk2[k]
                    tot = _take16(plsc.cumsum(acc0 + acc1), L - 1)
                    lvec = jnp.where(iota16 == j, lvec + tot, lvec)
                lvec = jnp.where(lvec > 0, lvec, 0.2 * lvec)
                lvec = jnp.where(rowidx < nb, lvec, _NEG)
                logits_v[pl.ds(c * L, L)] = lvec
                return jnp.maximum(mmax, lvec)

            mmax = lax.fori_loop(0, nch, p1_chunk, jnp.full((L,), _NEG, jnp.float32))
            mvec = _take16(plsc.cummax(mmax), L - 1)

            def pexp_chunk(c, ssum):
                lv = logits_v[pl.ds(c * L, L)]
                wv = jnp.exp(lv - mvec)
                w_v[pl.ds(c * L, L)] = wv
                return ssum + wv

            ssum = lax.fori_loop(0, nch, pexp_chunk, zero16)
            svec = _take16(plsc.cumsum(ssum), L - 1)

            def p2_chunk(c, accs):
                raccs, eaccs = accs
                wv = w_v[pl.ds(c * L, L)]
                nr = list(raccs)
                ne = list(eaccs)
                for j in range(L):
                    wj = _take16(wv, j)
                    n = c * L + j
                    for k in range(7):
                        nr[k] = nr[k] + wj * rel_rows[buf, n, pl.ds(k * L, L)]
                        ne[k] = ne[k] + wj * ent_rows[buf, n, pl.ds(k * L, L)]
                return tuple(nr), tuple(ne)

            zaccs = tuple(zero16 for _ in range(7))
            raccs, eaccs = lax.fori_loop(0, nch, p2_chunk, (zaccs, zaccs))
            for k in range(7):
                r_out_v[r, pl.ds(k * L, L)] = raccs[k] / svec
                e_out_v[r, pl.ds(k * L, L)] = eaccs[k] / svec

        def pair_body(p, c2):
            for parity in range(2):
                r = p * 2 + parity
                if parity == 0:
                    issue(r + 1, 1)
                else:
                    @pl.when(p < GRP // 2 - 1)
                    def _():
                        issue(r + 1, 0)
                wait(r, parity)
                compute_row(r, parity)
            return c2

        lax.fori_loop(0, GRP // 2, pair_body, 0)
        pltpu.sync_copy(r_out_v, r_out.at[pl.ds(gbase, GRP)])
        pltpu.sync_copy(e_out_v, e_out.at[pl.ds(gbase, GRP)])
        return carry

    lax.fori_loop(0, NGRP, group_body, 0)
    for h in range(4):
        cp = pltpu.make_async_copy(
            sym.at[selfidx_v.at[pl.ds(h * 32, 32)]], self_rows, sem_self)
        cp.start()
        cp.wait()
        pltpu.sync_copy(self_rows, self_out.at[pl.ds(base + h * 32, 32)])


_sc_mesh = plsc.VectorSubcoreMesh(core_axis_name="c", subcore_axis_name="s")

_sc_call = pl.kernel(
    _sc_body, mesh=_sc_mesh,
    compiler_params=pltpu.CompilerParams(needs_layout_passes=False,
                                         use_tc_tiling_on_sc=False),
    out_type=[
        jax.ShapeDtypeStruct((B, DP), jnp.float32),
        jax.ShapeDtypeStruct((B, DP), jnp.float32),
        jax.ShapeDtypeStruct((B, TW), jnp.float32),
    ],
    scratch_types=[
        pltpu.VMEM((GRP, 2, 100), jnp.int32),
        pltpu.VMEM((GRP, 2, 100), jnp.int32),
        pltpu.VMEM((RPW,), jnp.int32),
        pltpu.VMEM((RPW,), jnp.int32),
        pltpu.VMEM((32, TW), jnp.float32),
        pltpu.VMEM((2, NBP, TW), jnp.float32),
        pltpu.VMEM((2, NBP, TW), jnp.float32),
        pltpu.VMEM((GRP, DP), jnp.float32),
        pltpu.VMEM((GRP, DP), jnp.float32),
        pltpu.VMEM((8 * L,), jnp.float32),
        pltpu.VMEM((8 * L,), jnp.float32),
        pltpu.VMEM((NBP,), jnp.float32),
        pltpu.VMEM((NBP,), jnp.float32),
        pltpu.SemaphoreType.DMA,
        pltpu.SemaphoreType.DMA,
        pltpu.SemaphoreType.DMA,
        pltpu.SemaphoreType.DMA,
        pltpu.SemaphoreType.DMA,
    ],
)


PBLK = 4096
VP1 = 500001
PGRID = (VP1 + PBLK - 1) // PBLK


def _pad_body(xt, o):
    o[...] = jnp.concatenate(
        [xt[...].T, jnp.zeros((PBLK, TW - D), jnp.float32)], axis=1)


def _pad_table(symbol_emb):
    return pl.pallas_call(
        _pad_body,
        grid=(PGRID,),
        in_specs=[pl.BlockSpec((D, PBLK), lambda i: (0, i))],
        out_specs=pl.BlockSpec((PBLK, TW), lambda i: (i, 0)),
        out_shape=jax.ShapeDtypeStruct((VP1, TW), jnp.float32),
    )(symbol_emb.T)


BLK = 512


def _tc_body(r112, e112, selfe, w1p, w2p, bias, wg, cvec, out):
    bg = cvec[0, 0]
    agg = (jnp.dot(r112[...], w1p[...], preferred_element_type=jnp.float32)
           + jnp.dot(e112[...], w2p[...], preferred_element_type=jnp.float32) + bias[0])
    glog = jnp.sum(agg * wg[...], axis=1, keepdims=True)
    g = jax.nn.sigmoid(glog + bg)
    out[...] = jnp.tanh(g * agg + (1.0 - g) * selfe[..., :D])


def kernel(connections, num_neighbors, symbol_emb, W_gcn, b_gcn, gcn_b, W_attn, b_attn, W_gate, b_gate, gate_b):
    relidx = connections[:, :, 1].astype(jnp.int32).reshape(B, 2, 100)
    entidx = connections[:, :, 2].astype(jnp.int32).reshape(B, 2, 100)
    selfidx = connections[:, 0, 0].astype(jnp.int32)
    nnb = jnp.maximum(num_neighbors, 1).astype(jnp.int32)
    symp = _pad_table(symbol_emb)

    u = (W_attn @ W_gcn)[0]
    c = W_attn[0] @ (b_gcn + gcn_b) + b_attn[0]
    u1p = jnp.concatenate([u[:D], jnp.zeros((DP - D,), jnp.float32),
                           jnp.full((L,), c, jnp.float32)])
    u2p = jnp.concatenate([u[D:], jnp.zeros((28,), jnp.float32)])

    r112, e112, self128 = _sc_call(symp, relidx, entidx, selfidx, nnb, u1p, u2p)

    w1p = jnp.concatenate([W_gcn[:, :D].T, jnp.zeros((DP - D, D), jnp.float32)], axis=0)
    w2p = jnp.concatenate([W_gcn[:, D:].T, jnp.zeros((DP - D, D), jnp.float32)], axis=0)
    bias = (b_gcn + gcn_b)[None, :]
    cvec = jnp.stack([b_gate[0] + gate_b[0], 0.0])[None, :]

    grid = B // BLK
    out = pl.pallas_call(
        _tc_body,
        grid=(grid,),
        in_specs=[
            pl.BlockSpec((BLK, DP), lambda i: (i, 0)),
            pl.BlockSpec((BLK, DP), lambda i: (i, 0)),
            pl.BlockSpec((BLK, TW), lambda i: (i, 0)),
            pl.BlockSpec((DP, D), lambda i: (0, 0)),
            pl.BlockSpec((DP, D), lambda i: (0, 0)),
            pl.BlockSpec((1, D), lambda i: (0, 0)),
            pl.BlockSpec((1, D), lambda i: (0, 0)),
            pl.BlockSpec((1, 2), lambda i: (0, 0)),
        ],
        out_specs=pl.BlockSpec((BLK, D), lambda i: (i, 0)),
        out_shape=jax.ShapeDtypeStruct((B, D), jnp.float32),
    )(r112, e112, self128, w1p, w2p, bias, W_gate, cvec)
    return out


# speedup vs baseline: 3.0453x; 1.0159x over previous
import functools
import jax
import jax.numpy as jnp
from jax import lax
from jax.experimental import pallas as pl
from jax.experimental.pallas import tpu as pltpu
from jax.experimental.pallas import tpu_sc as plsc

B = 4096
NB = 200
D = 100
L = 16
NBP = 208
DP = 112
TW = 128
NW = 32
RPW = B // NW
GRP = 16
NGRP = RPW // GRP

_NEG = -1e9

_GDN = lax.GatherDimensionNumbers(offset_dims=(), collapsed_slice_dims=(0,),
                                  start_index_map=(0,))


def _take16(vec, j):
    idx = jnp.full((L, 1), j, jnp.int32)
    return lax.gather(vec, idx, _GDN, (1,),
                      mode=lax.GatherScatterMode.PROMISE_IN_BOUNDS)


def _sc_body(sym, relidx, entidx, selfidx_h, nnb_h, u1_h, u2_h,
             r_out, e_out, self_out,
             relidx_v, entidx_v, nb_v, selfidx_v, self_rows,
             rel_rows, ent_rows, r_out_v, e_out_v, u1_v, u2_v,
             logits_v, w_v,
             sem_r0, sem_r1, sem_e0, sem_e1, sem_self):
    wid = lax.axis_index("s") * 2 + lax.axis_index("c")
    base = wid * RPW
    pltpu.sync_copy(u1_h, u1_v)
    pltpu.sync_copy(u2_h, u2_v)
    pltpu.sync_copy(nnb_h.at[pl.ds(base, RPW)], nb_v)
    pltpu.sync_copy(selfidx_h.at[pl.ds(base, RPW)], selfidx_v)

    zero16 = jnp.zeros((L,), jnp.float32)
    sem_r = (sem_r0, sem_r1)
    sem_e = (sem_e0, sem_e1)

    def nch_of(row_local):
        rl = jnp.minimum(row_local, RPW - 1)
        nbch = nb_v[pl.ds((rl // L) * L, L)]
        nbvec = _take16(nbch, rl % L)
        return (nbvec[0] + (L - 1)) >> 4

    def issue(r, buf, nch):
        def body(c, carry):
            pltpu.make_async_copy(
                sym.at[relidx_v.at[r, c]],
                rel_rows.at[buf, pl.ds(c * L, L)], sem_r[buf]).start()
            pltpu.make_async_copy(
                sym.at[entidx_v.at[r, c]],
                ent_rows.at[buf, pl.ds(c * L, L)], sem_e[buf]).start()
            return carry
        lax.fori_loop(0, nch, body, 0)

    def wait(r, buf, nch):
        def body(c, carry):
            pltpu.make_async_copy(
                sym.at[relidx_v.at[r, c]],
                rel_rows.at[buf, pl.ds(c * L, L)], sem_r[buf]).wait()
            pltpu.make_async_copy(
                sym.at[entidx_v.at[r, c]],
                ent_rows.at[buf, pl.ds(c * L, L)], sem_e[buf]).wait()
            return carry
        lax.fori_loop(0, nch, body, 0)

    iota16 = lax.iota(jnp.int32, L)
    cvec = u1_v[pl.ds(DP, L)]

    def group_body(g, carry):
        gbase = base + g * GRP
        pltpu.sync_copy(relidx.at[pl.ds(gbase, GRP)], relidx_v)
        pltpu.sync_copy(entidx.at[pl.ds(gbase, GRP)], entidx_v)
        issue(0, 0, nch_of(g * GRP))

        def compute_row(r, buf):
            row_local = g * GRP + r
            nbch = nb_v[pl.ds((row_local // L) * L, L)]
            nbvec = _take16(nbch, row_local % L)
            nb = nbvec[0]
            nch = (nb + (L - 1)) >> 4

            def p1_chunk(c, mmax):
                rowidx = iota16 + c * L
                uk1 = [u1_v[pl.ds(k * L, L)] for k in range(7)]
                uk2 = [u2_v[pl.ds(k * L, L)] for k in range(7)]
                lvec = cvec
                for j in range(L):
                    n = c * L + j
                    acc0 = zero16
                    acc1 = zero16
                    for k in range(7):
                        acc0 = acc0 + rel_rows[buf, n, pl.ds(k * L, L)] * uk1[k]
                        acc1 = acc1 + ent_rows[buf, n, pl.ds(k * L, L)] * uk2[k]
                    tot = _take16(plsc.cumsum(acc0 + acc1), L - 1)
                    lvec = jnp.where(iota16 == j, lvec + tot, lvec)
                lvec = jnp.where(lvec > 0, lvec, 0.2 * lvec)
                lvec = jnp.where(rowidx < nb, lvec, _NEG)
                logits_v[pl.ds(c * L, L)] = lvec
                return jnp.maximum(mmax, lvec)

            mmax = lax.fori_loop(0, nch, p1_chunk, jnp.full((L,), _NEG, jnp.float32))
            mvec = _take16(plsc.cummax(mmax), L - 1)

            def pexp_chunk(c, ssum):
                lv = logits_v[pl.ds(c * L, L)]
                wv = jnp.exp(lv - mvec)
                w_v[pl.ds(c * L, L)] = wv
                return ssum + wv

            ssum = lax.fori_loop(0, nch, pexp_chunk, zero16)
            svec = _take16(plsc.cumsum(ssum), L - 1)

            def p2_chunk(c, accs):
                raccs, eaccs = accs
                wv = w_v[pl.ds(c * L, L)]
                nr = list(raccs)
                ne = list(eaccs)
                for j in range(L):
                    wj = _take16(wv, j)
                    n = c * L + j
                    for k in range(7):
                        nr[k] = nr[k] + wj * rel_rows[buf, n, pl.ds(k * L, L)]
                        ne[k] = ne[k] + wj * ent_rows[buf, n, pl.ds(k * L, L)]
                return tuple(nr), tuple(ne)

            zaccs = tuple(zero16 for _ in range(7))
            raccs, eaccs = lax.fori_loop(0, nch, p2_chunk, (zaccs, zaccs))
            for k in range(7):
                r_out_v[r, pl.ds(k * L, L)] = raccs[k] / svec
                e_out_v[r, pl.ds(k * L, L)] = eaccs[k] / svec

        def pair_body(p, c2):
            for parity in range(2):
                r = p * 2 + parity
                nch_r = nch_of(g * GRP + r)
                nch_n = nch_of(g * GRP + r + 1)
                if parity == 0:
                    issue(r + 1, 1, nch_n)
                else:
                    @pl.when(p < GRP // 2 - 1)
                    def _():
                        issue(r + 1, 0, nch_n)
                wait(r, parity, nch_r)
                compute_row(r, parity)
            return c2

        lax.fori_loop(0, GRP // 2, pair_body, 0)
        pltpu.sync_copy(r_out_v, r_out.at[pl.ds(gbase, GRP)])
        pltpu.sync_copy(e_out_v, e_out.at[pl.ds(gbase, GRP)])
        return carry

    lax.fori_loop(0, NGRP, group_body, 0)
    for h in range(4):
        cp = pltpu.make_async_copy(
            sym.at[selfidx_v.at[pl.ds(h * 32, 32)]], self_rows, sem_self)
        cp.start()
        cp.wait()
        pltpu.sync_copy(self_rows, self_out.at[pl.ds(base + h * 32, 32)])


_sc_mesh = plsc.VectorSubcoreMesh(core_axis_name="c", subcore_axis_name="s")

_sc_call = pl.kernel(
    _sc_body, mesh=_sc_mesh,
    compiler_params=pltpu.CompilerParams(needs_layout_passes=False,
                                         use_tc_tiling_on_sc=False),
    out_type=[
        jax.ShapeDtypeStruct((B, DP), jnp.float32),
        jax.ShapeDtypeStruct((B, DP), jnp.float32),
        jax.ShapeDtypeStruct((B, TW), jnp.float32),
    ],
    scratch_types=[
        pltpu.VMEM((GRP, 13, L), jnp.int32),
        pltpu.VMEM((GRP, 13, L), jnp.int32),
        pltpu.VMEM((RPW,), jnp.int32),
        pltpu.VMEM((RPW,), jnp.int32),
        pltpu.VMEM((32, TW), jnp.float32),
        pltpu.VMEM((2, NBP, TW), jnp.float32),
        pltpu.VMEM((2, NBP, TW), jnp.float32),
        pltpu.VMEM((GRP, DP), jnp.float32),
        pltpu.VMEM((GRP, DP), jnp.float32),
        pltpu.VMEM((8 * L,), jnp.float32),
        pltpu.VMEM((8 * L,), jnp.float32),
        pltpu.VMEM((NBP,), jnp.float32),
        pltpu.VMEM((NBP,), jnp.float32),
        pltpu.SemaphoreType.DMA,
        pltpu.SemaphoreType.DMA,
        pltpu.SemaphoreType.DMA,
        pltpu.SemaphoreType.DMA,
        pltpu.SemaphoreType.DMA,
    ],
)


PBLK = 4096
VP1 = 500001
PGRID = (VP1 + PBLK - 1) // PBLK


def _pad_body(xt, o):
    o[...] = jnp.concatenate(
        [xt[...].T, jnp.zeros((PBLK, TW - D), jnp.float32)], axis=1)


def _pad_table(symbol_emb):
    return pl.pallas_call(
        _pad_body,
        grid=(PGRID,),
        in_specs=[pl.BlockSpec((D, PBLK), lambda i: (0, i))],
        out_specs=pl.BlockSpec((PBLK, TW), lambda i: (i, 0)),
        out_shape=jax.ShapeDtypeStruct((VP1, TW), jnp.float32),
    )(symbol_emb.T)


BLK = 512


def _tc_body(r112, e112, selfe, w1p, w2p, bias, wg, cvec, out):
    bg = cvec[0, 0]
    agg = (jnp.dot(r112[...], w1p[...], preferred_element_type=jnp.float32)
           + jnp.dot(e112[...], w2p[...], preferred_element_type=jnp.float32) + bias[0])
    glog = jnp.sum(agg * wg[...], axis=1, keepdims=True)
    g = jax.nn.sigmoid(glog + bg)
    out[...] = jnp.tanh(g * agg + (1.0 - g) * selfe[..., :D])


def kernel(connections, num_neighbors, symbol_emb, W_gcn, b_gcn, gcn_b, W_attn, b_attn, W_gate, b_gate, gate_b):
    relidx = jnp.pad(connections[:, :, 1].astype(jnp.int32),
                     ((0, 0), (0, NBP - NB))).reshape(B, 13, L)
    entidx = jnp.pad(connections[:, :, 2].astype(jnp.int32),
                     ((0, 0), (0, NBP - NB))).reshape(B, 13, L)
    selfidx = connections[:, 0, 0].astype(jnp.int32)
    nnb = jnp.maximum(num_neighbors, 1).astype(jnp.int32)
    symp = _pad_table(symbol_emb)

    u = (W_attn @ W_gcn)[0]
    c = W_attn[0] @ (b_gcn + gcn_b) + b_attn[0]
    u1p = jnp.concatenate([u[:D], jnp.zeros((DP - D,), jnp.float32),
                           jnp.full((L,), c, jnp.float32)])
    u2p = jnp.concatenate([u[D:], jnp.zeros((28,), jnp.float32)])

    r112, e112, self128 = _sc_call(symp, relidx, entidx, selfidx, nnb, u1p, u2p)

    w1p = jnp.concatenate([W_gcn[:, :D].T, jnp.zeros((DP - D, D), jnp.float32)], axis=0)
    w2p = jnp.concatenate([W_gcn[:, D:].T, jnp.zeros((DP - D, D), jnp.float32)], axis=0)
    bias = (b_gcn + gcn_b)[None, :]
    cvec = jnp.stack([b_gate[0] + gate_b[0], 0.0])[None, :]

    grid = B // BLK
    out = pl.pallas_call(
        _tc_body,
        grid=(grid,),
        in_specs=[
            pl.BlockSpec((BLK, DP), lambda i: (i, 0)),
            pl.BlockSpec((BLK, DP), lambda i: (i, 0)),
            pl.BlockSpec((BLK, TW), lambda i: (i, 0)),
            pl.BlockSpec((DP, D), lambda i: (0, 0)),
            pl.BlockSpec((DP, D), lambda i: (0, 0)),
            pl.BlockSpec((1, D), lambda i: (0, 0)),
            pl.BlockSpec((1, D), lambda i: (0, 0)),
            pl.BlockSpec((1, 2), lambda i: (0, 0)),
        ],
        out_specs=pl.BlockSpec((BLK, D), lambda i: (i, 0)),
        out_shape=jax.ShapeDtypeStruct((B, D), jnp.float32),
    )(r112, e112, self128, w1p, w2p, bias, W_gate, cvec)
    return out


# speedup vs baseline: 3.2893x; 1.0801x over previous
import functools
import jax
import jax.numpy as jnp
from jax import lax
from jax.experimental import pallas as pl
from jax.experimental.pallas import tpu as pltpu
from jax.experimental.pallas import tpu_sc as plsc

B = 4096
NB = 200
D = 100
L = 16
NBP = 208
DP = 112
TW = 128
NW = 32
RPW = B // NW
GRP = 16
NGRP = RPW // GRP

_NEG = -1e9

_GDN = lax.GatherDimensionNumbers(offset_dims=(), collapsed_slice_dims=(0,),
                                  start_index_map=(0,))


def _take16(vec, j):
    idx = jnp.full((L, 1), j, jnp.int32)
    return lax.gather(vec, idx, _GDN, (1,),
                      mode=lax.GatherScatterMode.PROMISE_IN_BOUNDS)


def _sc_body(sym, relidx, entidx, selfidx_h, nnb_h, u1_h, u2_h,
             r_out, e_out, self_out,
             relidx_v, entidx_v, nb_v, selfidx_v, self_rows,
             rel_rows, ent_rows, r_out_v, e_out_v, u1_v, u2_v,
             logits_v, w_v,
             sem_r0, sem_r1, sem_e0, sem_e1, sem_self):
    wid = lax.axis_index("s") * 2 + lax.axis_index("c")
    base = wid * RPW
    pltpu.sync_copy(u1_h, u1_v)
    pltpu.sync_copy(u2_h, u2_v)
    pltpu.sync_copy(nnb_h.at[pl.ds(base, RPW)], nb_v)
    pltpu.sync_copy(selfidx_h.at[pl.ds(base, RPW)], selfidx_v)

    zero16 = jnp.zeros((L,), jnp.float32)
    sem_r = (sem_r0, sem_r1)
    sem_e = (sem_e0, sem_e1)

    def nch_of(row_local):
        rl = jnp.minimum(row_local, RPW - 1)
        nbch = nb_v[pl.ds((rl // L) * L, L)]
        nbvec = _take16(nbch, rl % L)
        return (nbvec[0] + (L - 1)) >> 4

    def issue(r, buf, nch):
        def body(c, carry):
            pltpu.make_async_copy(
                sym.at[relidx_v.at[r, c]],
                rel_rows.at[buf, pl.ds(c * L, L)], sem_r[buf]).start()
            pltpu.make_async_copy(
                sym.at[entidx_v.at[r, c]],
                ent_rows.at[buf, pl.ds(c * L, L)], sem_e[buf]).start()
            return carry
        lax.fori_loop(0, nch, body, 0)

    def wait(r, buf, nch):
        def body(c, carry):
            pltpu.make_async_copy(
                sym.at[relidx_v.at[r, c]],
                rel_rows.at[buf, pl.ds(c * L, L)], sem_r[buf]).wait()
            pltpu.make_async_copy(
                sym.at[entidx_v.at[r, c]],
                ent_rows.at[buf, pl.ds(c * L, L)], sem_e[buf]).wait()
            return carry
        lax.fori_loop(0, nch, body, 0)

    iota16 = lax.iota(jnp.int32, L)
    cvec = u1_v[pl.ds(DP, L)]
    uk1 = [u1_v[pl.ds(k * L, L)] for k in range(7)]
    uk2 = [u2_v[pl.ds(k * L, L)] for k in range(7)]

    def group_body(g, carry):
        gbase = base + g * GRP
        pltpu.sync_copy(relidx.at[pl.ds(gbase, GRP)], relidx_v)
        pltpu.sync_copy(entidx.at[pl.ds(gbase, GRP)], entidx_v)
        issue(0, 0, nch_of(g * GRP))

        def compute_row(r, buf):
            row_local = g * GRP + r
            nbch = nb_v[pl.ds((row_local // L) * L, L)]
            nbvec = _take16(nbch, row_local % L)
            nb = nbvec[0]
            nch = (nb + (L - 1)) >> 4

            def p1_chunk(c, mmax):
                rowidx = iota16 + c * L
                winr = rel_rows.at[buf, pl.ds(c * L, L)]
                wine = ent_rows.at[buf, pl.ds(c * L, L)]
                lvec = cvec
                for j in range(L):
                    acc0 = zero16
                    acc1 = zero16
                    for k in range(7):
                        acc0 = acc0 + winr[j, pl.ds(k * L, L)] * uk1[k]
                        acc1 = acc1 + wine[j, pl.ds(k * L, L)] * uk2[k]
                    tot = _take16(plsc.cumsum(acc0 + acc1), L - 1)
                    lvec = jnp.where(iota16 == j, lvec + tot, lvec)
                lvec = jnp.where(lvec > 0, lvec, 0.2 * lvec)
                lvec = jnp.where(rowidx < nb, lvec, _NEG)
                logits_v[pl.ds(c * L, L)] = lvec
                return jnp.maximum(mmax, lvec)

            mmax = lax.fori_loop(0, nch, p1_chunk, jnp.full((L,), _NEG, jnp.float32))
            mvec = _take16(plsc.cummax(mmax), L - 1)

            def p2_chunk(c, accs):
                raccs, eaccs, ssum = accs
                lv = logits_v[pl.ds(c * L, L)]
                wv = jnp.exp(lv - mvec)
                winr = rel_rows.at[buf, pl.ds(c * L, L)]
                wine = ent_rows.at[buf, pl.ds(c * L, L)]
                nr = list(raccs)
                ne = list(eaccs)
                for j in range(L):
                    wj = _take16(wv, j)
                    for k in range(7):
                        nr[k] = nr[k] + wj * winr[j, pl.ds(k * L, L)]
                        ne[k] = ne[k] + wj * wine[j, pl.ds(k * L, L)]
                return tuple(nr), tuple(ne), ssum + wv

            zaccs = tuple(zero16 for _ in range(7))
            raccs, eaccs, ssum = lax.fori_loop(
                0, nch, p2_chunk, (zaccs, zaccs, zero16))
            svec = _take16(plsc.cumsum(ssum), L - 1)
            for k in range(7):
                r_out_v[r, pl.ds(k * L, L)] = raccs[k] / svec
                e_out_v[r, pl.ds(k * L, L)] = eaccs[k] / svec

        def pair_body(p, c2):
            for parity in range(2):
                r = p * 2 + parity
                nch_r = nch_of(g * GRP + r)
                nch_n = nch_of(g * GRP + r + 1)
                if parity == 0:
                    issue(r + 1, 1, nch_n)
                else:
                    @pl.when(p < GRP // 2 - 1)
                    def _():
                        issue(r + 1, 0, nch_n)
                wait(r, parity, nch_r)
                compute_row(r, parity)
            return c2

        lax.fori_loop(0, GRP // 2, pair_body, 0)
        pltpu.sync_copy(r_out_v, r_out.at[pl.ds(gbase, GRP)])
        pltpu.sync_copy(e_out_v, e_out.at[pl.ds(gbase, GRP)])
        return carry

    lax.fori_loop(0, NGRP, group_body, 0)
    for h in range(4):
        cp = pltpu.make_async_copy(
            sym.at[selfidx_v.at[pl.ds(h * 32, 32)]], self_rows, sem_self)
        cp.start()
        cp.wait()
        pltpu.sync_copy(self_rows, self_out.at[pl.ds(base + h * 32, 32)])


_sc_mesh = plsc.VectorSubcoreMesh(core_axis_name="c", subcore_axis_name="s")

_sc_call = pl.kernel(
    _sc_body, mesh=_sc_mesh,
    compiler_params=pltpu.CompilerParams(needs_layout_passes=False,
                                         use_tc_tiling_on_sc=False),
    out_type=[
        jax.ShapeDtypeStruct((B, DP), jnp.float32),
        jax.ShapeDtypeStruct((B, DP), jnp.float32),
        jax.ShapeDtypeStruct((B, TW), jnp.float32),
    ],
    scratch_types=[
        pltpu.VMEM((GRP, 13, L), jnp.int32),
        pltpu.VMEM((GRP, 13, L), jnp.int32),
        pltpu.VMEM((RPW,), jnp.int32),
        pltpu.VMEM((RPW,), jnp.int32),
        pltpu.VMEM((32, TW), jnp.float32),
        pltpu.VMEM((2, NBP, TW), jnp.float32),
        pltpu.VMEM((2, NBP, TW), jnp.float32),
        pltpu.VMEM((GRP, DP), jnp.float32),
        pltpu.VMEM((GRP, DP), jnp.float32),
        pltpu.VMEM((8 * L,), jnp.float32),
        pltpu.VMEM((8 * L,), jnp.float32),
        pltpu.VMEM((NBP,), jnp.float32),
        pltpu.VMEM((NBP,), jnp.float32),
        pltpu.SemaphoreType.DMA,
        pltpu.SemaphoreType.DMA,
        pltpu.SemaphoreType.DMA,
        pltpu.SemaphoreType.DMA,
        pltpu.SemaphoreType.DMA,
    ],
)


PBLK = 4096
VP1 = 500001
PGRID = (VP1 + PBLK - 1) // PBLK


def _pad_body(xt, o):
    o[...] = jnp.concatenate(
        [xt[...].T, jnp.zeros((PBLK, TW - D), jnp.float32)], axis=1)


def _pad_table(symbol_emb):
    return pl.pallas_call(
        _pad_body,
        grid=(PGRID,),
        in_specs=[pl.BlockSpec((D, PBLK), lambda i: (0, i))],
        out_specs=pl.BlockSpec((PBLK, TW), lambda i: (i, 0)),
        out_shape=jax.ShapeDtypeStruct((VP1, TW), jnp.float32),
    )(symbol_emb.T)


BLK = 512


def _tc_body(r112, e112, selfe, w1p, w2p, bias, wg, cvec, out):
    bg = cvec[0, 0]
    agg = (jnp.dot(r112[...], w1p[...], preferred_element_type=jnp.float32)
           + jnp.dot(e112[...], w2p[...], preferred_element_type=jnp.float32) + bias[0])
    glog = jnp.sum(agg * wg[...], axis=1, keepdims=True)
    g = jax.nn.sigmoid(glog + bg)
    out[...] = jnp.tanh(g * agg + (1.0 - g) * selfe[..., :D])


def kernel(connections, num_neighbors, symbol_emb, W_gcn, b_gcn, gcn_b, W_attn, b_attn, W_gate, b_gate, gate_b):
    relidx = jnp.pad(connections[:, :, 1].astype(jnp.int32),
                     ((0, 0), (0, NBP - NB))).reshape(B, 13, L)
    entidx = jnp.pad(connections[:, :, 2].astype(jnp.int32),
                     ((0, 0), (0, NBP - NB))).reshape(B, 13, L)
    selfidx = connections[:, 0, 0].astype(jnp.int32)
    nnb = jnp.maximum(num_neighbors, 1).astype(jnp.int32)
    symp = _pad_table(symbol_emb)

    u = (W_attn @ W_gcn)[0]
    c = W_attn[0] @ (b_gcn + gcn_b) + b_attn[0]
    u1p = jnp.concatenate([u[:D], jnp.zeros((DP - D,), jnp.float32),
                           jnp.full((L,), c, jnp.float32)])
    u2p = jnp.concatenate([u[D:], jnp.zeros((28,), jnp.float32)])

    r112, e112, self128 = _sc_call(symp, relidx, entidx, selfidx, nnb, u1p, u2p)

    w1p = jnp.concatenate([W_gcn[:, :D].T, jnp.zeros((DP - D, D), jnp.float32)], axis=0)
    w2p = jnp.concatenate([W_gcn[:, D:].T, jnp.zeros((DP - D, D), jnp.float32)], axis=0)
    bias = (b_gcn + gcn_b)[None, :]
    cvec = jnp.stack([b_gate[0] + gate_b[0], 0.0])[None, :]

    grid = B // BLK
    out = pl.pallas_call(
        _tc_body,
        grid=(grid,),
        in_specs=[
            pl.BlockSpec((BLK, DP), lambda i: (i, 0)),
            pl.BlockSpec((BLK, DP), lambda i: (i, 0)),
            pl.BlockSpec((BLK, TW), lambda i: (i, 0)),
            pl.BlockSpec((DP, D), lambda i: (0, 0)),
            pl.BlockSpec((DP, D), lambda i: (0, 0)),
            pl.BlockSpec((1, D), lambda i: (0, 0)),
            pl.BlockSpec((1, D), lambda i: (0, 0)),
            pl.BlockSpec((1, 2), lambda i: (0, 0)),
        ],
        out_specs=pl.BlockSpec((BLK, D), lambda i: (i, 0)),
        out_shape=jax.ShapeDtypeStruct((B, D), jnp.float32),
    )(r112, e112, self128, w1p, w2p, bias, W_gate, cvec)
    return out
